# baseline (speedup 1.0000x reference)
import functools

import jax
import jax.numpy as jnp
from jax import lax
from jax.experimental import pallas as pl
from jax.experimental.pallas import tpu as pltpu
from jax.experimental.pallas import tpu_sc as plsc

N = 10000
E = 320000
D_IN = 128
H = 48

NC = 2
NS = 16
NW = NC * NS
CH = 128
NITER = 80
EPT = NITER * CH
EP = NW * EPT
NP = 10240
RPS = NP // NS
LCAP = EPT + 16
LCAP2 = EPT + 128
FR = NP // 16

_CPARAMS = dict(needs_layout_passes=False, use_tc_tiling_on_sc=False)


def _vmesh():
    return plsc.VectorSubcoreMesh(
        core_axis_name="c", subcore_axis_name="s",
        num_cores=NC, num_subcores=NS)


@functools.lru_cache(maxsize=None)
def _make_sc_degscan():
  @functools.partial(
      pl.kernel,
      out_type=[
          jax.ShapeDtypeStruct((NW, NP), jnp.float32),
          jax.ShapeDtypeStruct((NW, LCAP), jnp.int32),
          jax.ShapeDtypeStruct((NW, LCAP), jnp.int32),
          jax.ShapeDtypeStruct((NW, 16), jnp.int32),
      ],
      mesh=_vmesh(),
      compiler_params=pltpu.CompilerParams(**_CPARAMS),
      scratch_types=[
          pltpu.VMEM((NITER, CH), jnp.int32),
          pltpu.VMEM((NITER, CH), jnp.int32),
          pltpu.VMEM((NP,), jnp.float32),
          pltpu.VMEM((LCAP,), jnp.int32),
          pltpu.VMEM((LCAP,), jnp.int32),
          pltpu.VMEM((16,), jnp.int32),
          pltpu.VMEM((16,), jnp.int32),
      ],
  )
  def _sc_degscan_k(dst3_hbm, src3_hbm, outf_hbm,
                    degp_hbm, msrc_hbm, mj_hbm, cnt_hbm,
                    dst_v, src_v, deg_v, msrc_v, mj_v, outf_v, cntb_v):
    cid = lax.axis_index("c")
    sid = lax.axis_index("s")
    wid = sid * NC + cid

    def zero_body(i, _):
        deg_v[pl.ds(i * 16, 16)] = jnp.zeros((16,), jnp.float32)
        return 0
    lax.fori_loop(0, NP // 16, zero_body, 0)

    pltpu.sync_copy(dst3_hbm.at[wid], dst_v)
    pltpu.sync_copy(src3_hbm.at[wid], src_v)
    pltpu.sync_copy(outf_hbm, outf_v)

    ones = jnp.ones((16,), jnp.float32)
    lanes = lax.iota(jnp.int32, 16)
    ov = outf_v[...]
    outs = [jnp.sum(jnp.where(lanes == k, ov, 0)) for k in range(9)]

    def body(j, cnt):
        for c in range(CH // 16):
            dstv = dst_v[j, pl.ds(c * 16, 16)]
            plsc.addupdate_scatter(deg_v, [dstv], ones)
            jsel = jnp.full((16,), 15, jnp.int32)
            for k in range(8, -1, -1):
                jsel = jnp.where(dstv == outs[k], jnp.int32(k), jsel)
            mask = jsel < 15
            srcv = src_v[j, pl.ds(c * 16, 16)]
            plsc.store_compressed(msrc_v.at[pl.ds(cnt, 16)], srcv, mask=mask)
            plsc.store_compressed(mj_v.at[pl.ds(cnt, 16)], jsel, mask=mask)
            cnt = cnt + jnp.sum(mask.astype(jnp.int32))
        return cnt
    cnt = lax.fori_loop(0, NITER, body, jnp.int32(0))

    msrc_v[pl.ds(cnt, 16)] = jnp.full((16,), NP - 1, jnp.int32)
    cntb_v[...] = jnp.full((16,), cnt, jnp.int32)

    pltpu.sync_copy(deg_v, degp_hbm.at[wid])
    pltpu.sync_copy(msrc_v, msrc_hbm.at[wid])
    pltpu.sync_copy(mj_v, mj_hbm.at[wid])
    pltpu.sync_copy(cntb_v, cnt_hbm.at[wid])

  return _sc_degscan_k


def _sc_degscan(dst3, src3, outf16):
  return _make_sc_degscan()(dst3, src3, outf16)


@functools.lru_cache(maxsize=None)
def _make_sc_edge():
  @functools.partial(
      pl.kernel,
      out_type=jax.ShapeDtypeStruct((NC, NP, H), jnp.float32),
      mesh=_vmesh(),
      compiler_params=pltpu.CompilerParams(**_CPARAMS),
      scratch_types=[
          pltpu.VMEM((NITER, CH), jnp.int32),
          pltpu.VMEM((NITER, CH), jnp.int32),
          pltpu.VMEM((CH, H), jnp.float32),
          pltpu.VMEM((CH, H), jnp.float32),
          pltpu.VMEM_SHARED((NP, H), jnp.float32),
          pltpu.SemaphoreType.DMA,
          pltpu.SemaphoreType.DMA,
          pltpu.SemaphoreType.DMA,
          pltpu.SemaphoreType.DMA,
      ],
  )
  def _sc_edge_k(src3_hbm, dst3_hbm, table_hbm, zeros_hbm, out_hbm,
                 src_v, dst_v, rows0, rows1, acc_sh, sem0, sem1, semw0, semw1):
    cid = lax.axis_index("c")
    sid = lax.axis_index("s")
    wid = sid * NC + cid

    row0 = pl.multiple_of(sid * RPS, 8)
    pltpu.sync_copy(zeros_hbm.at[pl.ds(row0, RPS)],
                    acc_sh.at[pl.ds(row0, RPS)])
    pltpu.sync_copy(src3_hbm.at[wid], src_v)
    pltpu.sync_copy(dst3_hbm.at[wid], dst_v)
    plsc.subcore_barrier()

    def start_g(j, buf, sem):
        pltpu.async_copy(table_hbm.at[src_v.at[j]], buf, sem)

    def wait_g(j, buf, sem):
        pltpu.make_async_copy(table_hbm.at[src_v.at[j]], buf, sem).wait()

    start_g(0, rows0, sem0)
    start_g(1, rows1, sem1)

    def body(jj, _):
        j0 = jj * 2
        j1 = j0 + 1
        wait_g(j0, rows0, sem0)
        s0 = pltpu.async_copy(rows0, acc_sh.at[dst_v.at[j0]], semw0, add=True)
        wait_g(j1, rows1, sem1)
        s1 = pltpu.async_copy(rows1, acc_sh.at[dst_v.at[j1]], semw1, add=True)
        s0.wait()

        @pl.when(jj != NITER // 2 - 1)
        def _():
            start_g(j0 + 2, rows0, sem0)

        s1.wait()

        @pl.when(jj != NITER // 2 - 1)
        def _():
            start_g(j1 + 2, rows1, sem1)
        return 0
    lax.fori_loop(0, NITER // 2, body, 0)

    plsc.subcore_barrier()
    pltpu.sync_copy(acc_sh.at[pl.ds(row0, RPS)],
                    out_hbm.at[cid, pl.ds(row0, RPS)])

  return _sc_edge_k


def _sc_edge(src3, dst3, table, zeros):
  return _make_sc_edge()(src3, dst3, table, zeros)


@functools.lru_cache(maxsize=None)
def _make_sc_flag():
  @functools.partial(
      pl.kernel,
      out_type=jax.ShapeDtypeStruct((NC, FR, 16), jnp.float32),
      mesh=_vmesh(),
      compiler_params=pltpu.CompilerParams(**_CPARAMS),
      scratch_types=[
          pltpu.VMEM((LCAP,), jnp.int32),
          pltpu.VMEM((16,), jnp.int32),
          pltpu.VMEM((16,), jnp.int32),
          pltpu.VMEM((FR, 16), jnp.float32),
          pltpu.VMEM((5, 128), jnp.int32),
          pltpu.VMEM_SHARED((FR, 16), jnp.float32),
      ],
  )
  def _sc_flag_k(msrc_hbm, cnt_hbm, outf_hbm, zeros_hbm, flagp_hbm,
                 mlist_v, cnt_v, outf_v, flag_v, ident_v, flag_sh):
    cid = lax.axis_index("c")
    sid = lax.axis_index("s")
    wid = sid * NC + cid
    lanes = lax.iota(jnp.int32, 16)
    ones = jnp.ones((16,), jnp.float32)
    zeros16 = jnp.zeros((16,), jnp.float32)

    pltpu.sync_copy(msrc_hbm.at[wid], mlist_v)
    pltpu.sync_copy(cnt_hbm.at[wid], cnt_v)
    pltpu.sync_copy(outf_hbm, outf_v)

    def zb(i, _):
        flag_v[i, pl.ds(0, 16)] = zeros16
        return 0
    lax.fori_loop(0, FR, zb, 0)
    for s in range(40):
        ident_v[s // 8, pl.ds((s % 8) * 16, 16)] = jnp.int32(s * 16) + lanes

    srow = pl.multiple_of(sid * (FR // NS), 8)
    pltpu.sync_copy(zeros_hbm.at[pl.ds(srow, FR // NS), pl.ds(0, 16)],
                    flag_sh.at[pl.ds(srow, FR // NS)])

    cnt = jnp.max(cnt_v[...])
    nch = (cnt + 15) // 16

    def fb(c, _):
        base = pl.multiple_of(c * 16, 8)
        mv = mlist_v[pl.ds(base, 16)]
        valid = (lanes + base) < cnt
        mvs = jnp.where(valid, mv, 0)
        plsc.addupdate_scatter(flag_v, [mvs // 16, mvs % 16], ones,
                               mask=valid)
        return 0
    lax.fori_loop(0, nch, fb, 0)

    ov = outf_v[...]
    ovm = lanes < 9
    ovs = jnp.where(ovm, ov, 0)
    plsc.addupdate_scatter(flag_v, [ovs // 16, ovs % 16], ones, mask=ovm)

    plsc.subcore_barrier()
    for c in range(5):
        pltpu.sync_copy(flag_v.at[pl.ds(c * 128, 128)],
                        flag_sh.at[ident_v.at[c]], add=True)
    plsc.subcore_barrier()
    pltpu.sync_copy(flag_sh.at[pl.ds(srow, FR // NS)],
                    flagp_hbm.at[cid, pl.ds(srow, FR // NS)])

  return _sc_flag_k


def _sc_flag(msrc, cnts, outf16, zeros):
  return _make_sc_flag()(msrc, cnts, outf16, zeros)


@functools.lru_cache(maxsize=None)
def _make_sc_edge_sparse():
  @functools.partial(
      pl.kernel,
      out_type=jax.ShapeDtypeStruct((NC, NP, H), jnp.float32),
      mesh=_vmesh(),
      compiler_params=pltpu.CompilerParams(**_CPARAMS),
      scratch_types=[
          pltpu.VMEM((NITER, CH), jnp.int32),
          pltpu.VMEM((NITER, CH), jnp.int32),
          pltpu.VMEM((FR, 16), jnp.float32),
          pltpu.VMEM((FR, 16), jnp.float32),
          pltpu.VMEM((LCAP2,), jnp.int32),
          pltpu.VMEM((LCAP2,), jnp.int32),
          pltpu.VMEM((CH,), jnp.int32),
          pltpu.VMEM((CH,), jnp.int32),
          pltpu.VMEM((CH, H), jnp.float32),
          pltpu.VMEM_SHARED((NP, H), jnp.float32),
          pltpu.SemaphoreType.DMA,
      ],
  )
  def _sc_edge_sparse_k(src3_hbm, dst3_hbm, flagp_hbm, table_hbm, zeros_hbm,
                        out_hbm, src_v, dst_v, flag0_v, flag1_v, csrc_v,
                        cdst_v, sidx_v, didx_v, rows_v, acc_sh, sem):
    cid = lax.axis_index("c")
    sid = lax.axis_index("s")
    wid = sid * NC + cid
    lanes = lax.iota(jnp.int32, 16)

    row0 = pl.multiple_of(sid * RPS, 8)
    pltpu.sync_copy(zeros_hbm.at[pl.ds(row0, RPS)],
                    acc_sh.at[pl.ds(row0, RPS)])
    pltpu.sync_copy(src3_hbm.at[wid], src_v)
    pltpu.sync_copy(dst3_hbm.at[wid], dst_v)
    pltpu.sync_copy(flagp_hbm.at[0], flag0_v)
    pltpu.sync_copy(flagp_hbm.at[1], flag1_v)
    plsc.subcore_barrier()

    def mb(i, _):
        v = flag0_v[i, pl.ds(0, 16)] + flag1_v[i, pl.ds(0, 16)]
        flag0_v[i, pl.ds(0, 16)] = v
        return 0
    lax.fori_loop(0, FR, mb, 0)

    def scan(j, cnt):
        for c in range(CH // 16):
            dstv = dst_v[j, pl.ds(c * 16, 16)]
            fv = plsc.load_gather(flag0_v, [dstv // 16, dstv % 16])
            mask = fv > 0.0
            srcv = src_v[j, pl.ds(c * 16, 16)]
            plsc.store_compressed(csrc_v.at[pl.ds(cnt, 16)], srcv, mask=mask)
            plsc.store_compressed(cdst_v.at[pl.ds(cnt, 16)], dstv, mask=mask)
            cnt = cnt + jnp.sum(mask.astype(jnp.int32))
        return cnt
    cnt = lax.fori_loop(0, NITER, scan, jnp.int32(0))

    for t in range(CH // 16):
        csrc_v[pl.ds(cnt + t * 16, 16)] = jnp.full((16,), NP - 1, jnp.int32)
        cdst_v[pl.ds(cnt + t * 16, 16)] = (
            jnp.int32(N) + (jnp.int32(t * 16) + lanes) % jnp.int32(NP - N))

    nch = (cnt + CH - 1) // CH

    def gs(c, _):
        base = pl.multiple_of(c * CH, 8)
        for t in range(CH // 16):
            sidx_v[pl.ds(t * 16, 16)] = csrc_v[pl.ds(base + t * 16, 16)]
            didx_v[pl.ds(t * 16, 16)] = cdst_v[pl.ds(base + t * 16, 16)]
        pltpu.async_copy(table_hbm.at[sidx_v], rows_v, sem).wait()
        pltpu.sync_copy(rows_v, acc_sh.at[didx_v], add=True)
        return 0
    lax.fori_loop(0, nch, gs, 0)

    plsc.subcore_barrier()
    pltpu.sync_copy(acc_sh.at[pl.ds(row0, RPS)],
                    out_hbm.at[cid, pl.ds(row0, RPS)])

  return _sc_edge_sparse_k


def _sc_edge_sparse(src3, dst3, flagp, table, zeros):
  return _make_sc_edge_sparse()(src3, dst3, flagp, table, zeros)


@functools.lru_cache(maxsize=None)
def _make_sc_sparse():
  @functools.partial(
      pl.kernel,
      out_type=jax.ShapeDtypeStruct((NW, 16, H), jnp.float32),
      mesh=_vmesh(),
      compiler_params=pltpu.CompilerParams(**_CPARAMS),
      scratch_types=[
          pltpu.VMEM((LCAP,), jnp.int32),
          pltpu.VMEM((LCAP,), jnp.int32),
          pltpu.VMEM((16,), jnp.int32),
          pltpu.VMEM((16, H), jnp.float32),
          pltpu.VMEM((16, H), jnp.float32),
          pltpu.SemaphoreType.DMA,
      ],
  )
  def _sc_sparse_k(msrc_hbm, mj_hbm, cnt_hbm, table_hbm, out_hbm,
                   msrc_v, mj_v, cnt_v, buf_v, t_v, sem):
    cid = lax.axis_index("c")
    sid = lax.axis_index("s")
    wid = sid * NC + cid

    pltpu.sync_copy(msrc_hbm.at[wid], msrc_v)
    pltpu.sync_copy(mj_hbm.at[wid], mj_v)
    pltpu.sync_copy(cnt_hbm.at[wid], cnt_v)

    for r in range(16):
        for m in range(H // 16):
            t_v[r, pl.ds(m * 16, 16)] = jnp.zeros((16,), jnp.float32)

    cnt = jnp.max(cnt_v[...])
    nch = (cnt + 15) // 16
    lanes = lax.iota(jnp.int32, 16)

    def body(c, _):
        base = pl.multiple_of(c * 16, 8)
        pltpu.async_copy(
            table_hbm.at[msrc_v.at[pl.ds(base, 16)]], buf_v, sem).wait()
        jv = mj_v[pl.ds(base, 16)]
        valid = (lanes + base) < cnt
        for m in range(H):
            col = jnp.full((16,), m, jnp.int32)
            vm = plsc.load_gather(buf_v, [lanes, col])
            plsc.addupdate_scatter(t_v, [jv, col], vm, mask=valid)
        return 0
    lax.fori_loop(0, nch, body, 0)

    pltpu.sync_copy(t_v, out_hbm.at[wid])

  return _sc_sparse_k


def _sc_sparse(msrc, mj, cnts, table):
  return _make_sc_sparse()(msrc, mj, cnts, table)


def _tc1a_body(x_ref, w1_ref, p_ref):
    p_ref[...] = jnp.dot(x_ref[...], w1_ref[...],
                         preferred_element_type=jnp.float32)


def _tc1a(x, w1):
    blk = 2048
    grid = NP // blk
    return pl.pallas_call(
        _tc1a_body,
        grid=(grid,),
        in_specs=[
            pl.BlockSpec((blk, D_IN), lambda i: (i, 0)),
            pl.BlockSpec((D_IN, H), lambda i: (0, 0)),
        ],
        out_specs=pl.BlockSpec((blk, H), lambda i: (i, 0)),
        out_shape=jax.ShapeDtypeStruct((NP, H), jnp.float32),
    )(x, w1)


def _tc1b_body(degp_ref, p_ref, ptil_ref, dis_ref):
    i = pl.program_id(0)
    blk = p_ref.shape[0]
    deg = jnp.sum(degp_ref[:, pl.ds(i * blk, blk)], axis=0) + 1.0
    dis = lax.rsqrt(deg)
    ptil_ref[...] = p_ref[...] * dis[:, None]
    dis_ref[...] = dis[:, None]


def _tc1b(degp, p):
    blk = 2048
    grid = NP // blk
    return pl.pallas_call(
        _tc1b_body,
        grid=(grid,),
        in_specs=[
            pl.BlockSpec((NW, NP), lambda i: (0, 0)),
            pl.BlockSpec((blk, H), lambda i: (i, 0)),
        ],
        out_specs=[
            pl.BlockSpec((blk, H), lambda i: (i, 0)),
            pl.BlockSpec((blk, 1), lambda i: (i, 0)),
        ],
        out_shape=[
            jax.ShapeDtypeStruct((NP, H), jnp.float32),
            jax.ShapeDtypeStruct((NP, 1), jnp.float32),
        ],
    )(degp, p)


def _tc2_body(s_ref, ptil_ref, dis_ref, w2_ref, b1_ref, qtil_ref):
    dis = dis_ref[...]
    agg = dis * (s_ref[0] + s_ref[1] + ptil_ref[...]) + b1_ref[...]
    h1 = jnp.maximum(agg, 0.0)
    q = jnp.dot(h1, w2_ref[...], preferred_element_type=jnp.float32)
    qtil_ref[...] = q * dis


def _tc2(s1, ptil, dis, w2, b1):
    blk = 2048
    grid = NP // blk
    return pl.pallas_call(
        _tc2_body,
        grid=(grid,),
        in_specs=[
            pl.BlockSpec((NC, blk, H), lambda i: (0, i, 0)),
            pl.BlockSpec((blk, H), lambda i: (i, 0)),
            pl.BlockSpec((blk, 1), lambda i: (i, 0)),
            pl.BlockSpec((H, H), lambda i: (0, 0)),
            pl.BlockSpec((1, H), lambda i: (0, 0)),
        ],
        out_specs=pl.BlockSpec((blk, H), lambda i: (i, 0)),
        out_shape=jax.ShapeDtypeStruct((NP, H), jnp.float32),
    )(s1, ptil, dis, w2, b1)


def _tc3_body(outf_ref, tsum_ref, qtil_ref, dis_ref, b2_ref, wl_ref, bl_ref,
              out_ref, rows_ref):
    tred = jnp.sum(tsum_ref[...], axis=0)
    rows_ref[...] = jnp.zeros((16, H), jnp.float32)
    for j in range(9):
        idx = outf_ref[j]
        m = jnp.int32(j)
        for k in range(j - 1, -1, -1):
            m = jnp.where(outf_ref[k] == idx, jnp.int32(k), m)
        trow = jnp.zeros((1, H), jnp.float32)
        for k in range(j + 1):
            trow = trow + jnp.where(m == k, 1.0, 0.0) * tred[k:k + 1, :]
        qrow = qtil_ref[pl.ds(idx, 1), :]
        d = dis_ref[pl.ds(idx, 1), :]
        h2 = jnp.maximum(d * (trow + qrow) + b2_ref[...], 0.0)
        rows_ref[pl.ds(j, 1), :] = h2
    z = jnp.dot(rows_ref[...], wl_ref[...], preferred_element_type=jnp.float32)
    out_ref[...] = jax.nn.sigmoid(z + bl_ref[...])


def _tc3(outfall, tsum, qtil, dis, b2, wl, bl):
    return pl.pallas_call(
        _tc3_body,
        in_specs=[
            pl.BlockSpec(memory_space=pltpu.SMEM),
            pl.BlockSpec((NW, 16, H), lambda: (0, 0, 0)),
            pl.BlockSpec((NP, H), lambda: (0, 0)),
            pl.BlockSpec((NP, 1), lambda: (0, 0)),
            pl.BlockSpec((1, H), lambda: (0, 0)),
            pl.BlockSpec((H, 1), lambda: (0, 0)),
            pl.BlockSpec((1, 1), lambda: (0, 0)),
        ],
        out_specs=pl.BlockSpec((16, 1), lambda: (0, 0)),
        out_shape=jax.ShapeDtypeStruct((16, 1), jnp.float32),
        scratch_shapes=[pltpu.VMEM((16, H), jnp.float32)],
    )(outfall, tsum, qtil, dis, b2, wl, bl)


def kernel(x, edge_index, outfall_indices, W1, b1, W2, b2, Wl, bl):
    pad = N + (jnp.arange(EP - E, dtype=jnp.int32) % (NP - N))
    src3 = jnp.concatenate([edge_index[0], pad]).reshape(NW, NITER, CH)
    dst3 = jnp.concatenate([edge_index[1], pad]).reshape(NW, NITER, CH)
    zeros = jnp.zeros((NP, H), jnp.float32)
    x = jnp.concatenate([x, jnp.zeros((NP - N, D_IN), jnp.float32)], axis=0)
    outf16 = jnp.concatenate(
        [outfall_indices, jnp.full((7,), -1, jnp.int32)])

    p = _tc1a(x, W1)
    degp, msrc, mj, cnts = _sc_degscan(dst3, src3, outf16)
    flagp = _sc_flag(msrc, cnts, outf16, zeros)
    ptil, dis = _tc1b(degp, p)
    s1 = _sc_edge_sparse(src3, dst3, flagp, ptil, zeros)
    qtil = _tc2(s1, ptil, dis, W2, b1.reshape(1, H))
    tsum = _sc_sparse(msrc, mj, cnts, qtil)
    out = _tc3(outfall_indices, tsum, qtil, dis, b2.reshape(1, H),
               Wl, bl.reshape(1, 1))
    return out[:9, 0]


# speedup vs baseline: 1.2137x; 1.2137x over previous
import functools

import jax
import jax.numpy as jnp
from jax import lax
from jax.experimental import pallas as pl
from jax.experimental.pallas import tpu as pltpu
from jax.experimental.pallas import tpu_sc as plsc

N = 10000
E = 320000
D_IN = 128
H = 48

NC = 2
NS = 16
NW = NC * NS
CH = 128
NITER = 80
EPT = NITER * CH
EP = NW * EPT
NP = 10240
RPS = NP // NS
LCAP = EPT + 16

_CPARAMS = dict(needs_layout_passes=False, use_tc_tiling_on_sc=False)


def _vmesh():
    return plsc.VectorSubcoreMesh(
        core_axis_name="c", subcore_axis_name="s",
        num_cores=NC, num_subcores=NS)


@functools.lru_cache(maxsize=None)
def _make_sc_degscan():
  @functools.partial(
      pl.kernel,
      out_type=[
          jax.ShapeDtypeStruct((NW, NP), jnp.float32),
          jax.ShapeDtypeStruct((NW, LCAP), jnp.int32),
          jax.ShapeDtypeStruct((NW, LCAP), jnp.int32),
          jax.ShapeDtypeStruct((NW, 16), jnp.int32),
      ],
      mesh=_vmesh(),
      compiler_params=pltpu.CompilerParams(**_CPARAMS),
      scratch_types=[
          pltpu.VMEM((NITER, CH), jnp.int32),
          pltpu.VMEM((NITER, CH), jnp.int32),
          pltpu.VMEM((NP,), jnp.float32),
          pltpu.VMEM((LCAP,), jnp.int32),
          pltpu.VMEM((LCAP,), jnp.int32),
          pltpu.VMEM((16,), jnp.int32),
          pltpu.VMEM((16,), jnp.int32),
      ],
  )
  def _sc_degscan_k(dst3_hbm, src3_hbm, outf_hbm,
                    degp_hbm, msrc_hbm, mj_hbm, cnt_hbm,
                    dst_v, src_v, deg_v, msrc_v, mj_v, outf_v, cntb_v):
    cid = lax.axis_index("c")
    sid = lax.axis_index("s")
    wid = sid * NC + cid

    def zero_body(i, _):
        deg_v[pl.ds(i * 16, 16)] = jnp.zeros((16,), jnp.float32)
        return 0
    lax.fori_loop(0, NP // 16, zero_body, 0)

    pltpu.sync_copy(dst3_hbm.at[wid], dst_v)
    pltpu.sync_copy(src3_hbm.at[wid], src_v)
    pltpu.sync_copy(outf_hbm, outf_v)

    ones = jnp.ones((16,), jnp.float32)
    lanes = lax.iota(jnp.int32, 16)
    ov = outf_v[...]
    outs = [jnp.sum(jnp.where(lanes == k, ov, 0)) for k in range(9)]

    def body(j, cnt):
        for c in range(CH // 16):
            dstv = dst_v[j, pl.ds(c * 16, 16)]
            plsc.addupdate_scatter(deg_v, [dstv], ones)
            jsel = jnp.full((16,), 15, jnp.int32)
            for k in range(8, -1, -1):
                jsel = jnp.where(dstv == outs[k], jnp.int32(k), jsel)
            mask = jsel < 15
            srcv = src_v[j, pl.ds(c * 16, 16)]
            plsc.store_compressed(msrc_v.at[pl.ds(cnt, 16)], srcv, mask=mask)
            plsc.store_compressed(mj_v.at[pl.ds(cnt, 16)], jsel, mask=mask)
            cnt = cnt + jnp.sum(mask.astype(jnp.int32))
        return cnt
    cnt = lax.fori_loop(0, NITER, body, jnp.int32(0))

    msrc_v[pl.ds(cnt, 16)] = jnp.full((16,), NP - 1, jnp.int32)
    cntb_v[...] = jnp.full((16,), cnt, jnp.int32)

    pltpu.sync_copy(deg_v, degp_hbm.at[wid])
    pltpu.sync_copy(msrc_v, msrc_hbm.at[wid])
    pltpu.sync_copy(mj_v, mj_hbm.at[wid])
    pltpu.sync_copy(cntb_v, cnt_hbm.at[wid])

  return _sc_degscan_k


def _sc_degscan(dst3, src3, outf16):
  return _make_sc_degscan()(dst3, src3, outf16)


@functools.lru_cache(maxsize=None)
def _make_sc_edge():
  @functools.partial(
      pl.kernel,
      out_type=jax.ShapeDtypeStruct((NC, NP, H), jnp.float32),
      mesh=_vmesh(),
      compiler_params=pltpu.CompilerParams(**_CPARAMS),
      scratch_types=[
          pltpu.VMEM((NITER, CH), jnp.int32),
          pltpu.VMEM((NITER, CH), jnp.int32),
          pltpu.VMEM((CH, H), jnp.float32),
          pltpu.VMEM((CH, H), jnp.float32),
          pltpu.VMEM((CH, H), jnp.float32),
          pltpu.VMEM((CH, H), jnp.float32),
          pltpu.VMEM_SHARED((NP, H), jnp.float32),
          pltpu.SemaphoreType.DMA,
          pltpu.SemaphoreType.DMA,
          pltpu.SemaphoreType.DMA,
          pltpu.SemaphoreType.DMA,
          pltpu.SemaphoreType.DMA,
          pltpu.SemaphoreType.DMA,
          pltpu.SemaphoreType.DMA,
          pltpu.SemaphoreType.DMA,
      ],
  )
  def _sc_edge_k(src3_hbm, dst3_hbm, table_hbm, zeros_hbm, out_hbm,
                 src_v, dst_v, rows0, rows1, rows2, rows3, acc_sh,
                 sem0, sem1, sem2, sem3, semw0, semw1, semw2, semw3):
    cid = lax.axis_index("c")
    sid = lax.axis_index("s")
    wid = sid * NC + cid

    row0 = pl.multiple_of(sid * RPS, 8)
    pltpu.sync_copy(zeros_hbm.at[pl.ds(row0, RPS)],
                    acc_sh.at[pl.ds(row0, RPS)])
    pltpu.sync_copy(src3_hbm.at[wid], src_v)
    pltpu.sync_copy(dst3_hbm.at[wid], dst_v)
    plsc.subcore_barrier()

    def start_g(j, buf, sem):
        pltpu.async_copy(table_hbm.at[src_v.at[j]], buf, sem)

    def wait_g(j, buf, sem):
        pltpu.make_async_copy(table_hbm.at[src_v.at[j]], buf, sem).wait()

    bufs = (rows0, rows1, rows2, rows3)
    gsems = (sem0, sem1, sem2, sem3)
    wsems = (semw0, semw1, semw2, semw3)
    for b in range(4):
        start_g(b, bufs[b], gsems[b])

    def body(jj, _):
        js = [jj * 4 + b for b in range(4)]
        scs = []
        for b in range(4):
            wait_g(js[b], bufs[b], gsems[b])
            scs.append(pltpu.async_copy(
                bufs[b], acc_sh.at[dst_v.at[js[b]]], wsems[b], add=True))
        for b in range(4):
            scs[b].wait()

            @pl.when(jj != NITER // 4 - 1)
            def _():
                start_g(js[b] + 4, bufs[b], gsems[b])
        return 0
    lax.fori_loop(0, NITER // 4, body, 0)

    plsc.subcore_barrier()
    pltpu.sync_copy(acc_sh.at[pl.ds(row0, RPS)],
                    out_hbm.at[cid, pl.ds(row0, RPS)])

  return _sc_edge_k


def _sc_edge(src3, dst3, table, zeros):
  return _make_sc_edge()(src3, dst3, table, zeros)


@functools.lru_cache(maxsize=None)
def _make_sc_sparse():
  @functools.partial(
      pl.kernel,
      out_type=jax.ShapeDtypeStruct((NW, 16, H), jnp.float32),
      mesh=_vmesh(),
      compiler_params=pltpu.CompilerParams(**_CPARAMS),
      scratch_types=[
          pltpu.VMEM((LCAP,), jnp.int32),
          pltpu.VMEM((LCAP,), jnp.int32),
          pltpu.VMEM((16,), jnp.int32),
          pltpu.VMEM((16, H), jnp.float32),
          pltpu.VMEM((16, H), jnp.float32),
          pltpu.SemaphoreType.DMA,
      ],
  )
  def _sc_sparse_k(msrc_hbm, mj_hbm, cnt_hbm, table_hbm, out_hbm,
                   msrc_v, mj_v, cnt_v, buf_v, t_v, sem):
    cid = lax.axis_index("c")
    sid = lax.axis_index("s")
    wid = sid * NC + cid

    pltpu.sync_copy(msrc_hbm.at[wid], msrc_v)
    pltpu.sync_copy(mj_hbm.at[wid], mj_v)
    pltpu.sync_copy(cnt_hbm.at[wid], cnt_v)

    for r in range(16):
        for m in range(H // 16):
            t_v[r, pl.ds(m * 16, 16)] = jnp.zeros((16,), jnp.float32)

    cnt = jnp.max(cnt_v[...])
    nch = (cnt + 15) // 16
    lanes = lax.iota(jnp.int32, 16)

    def body(c, _):
        base = pl.multiple_of(c * 16, 8)
        pltpu.async_copy(
            table_hbm.at[msrc_v.at[pl.ds(base, 16)]], buf_v, sem).wait()
        jv = mj_v[pl.ds(base, 16)]
        valid = (lanes + base) < cnt
        for m in range(H):
            col = jnp.full((16,), m, jnp.int32)
            vm = plsc.load_gather(buf_v, [lanes, col])
            plsc.addupdate_scatter(t_v, [jv, col], vm, mask=valid)
        return 0
    lax.fori_loop(0, nch, body, 0)

    pltpu.sync_copy(t_v, out_hbm.at[wid])

  return _sc_sparse_k


def _sc_sparse(msrc, mj, cnts, table):
  return _make_sc_sparse()(msrc, mj, cnts, table)


def _tc1_body(degp_ref, x_ref, w1_ref, ptil_ref, dis_ref):
    i = pl.program_id(0)
    blk = x_ref.shape[0]
    deg = jnp.sum(degp_ref[:, pl.ds(i * blk, blk)], axis=0) + 1.0
    dis = lax.rsqrt(deg)
    p = jnp.dot(x_ref[...], w1_ref[...], preferred_element_type=jnp.float32)
    ptil_ref[...] = p * dis[:, None]
    dis_ref[...] = dis[:, None]


def _tc1(degp, x, w1):
    blk = 2048
    grid = NP // blk
    return pl.pallas_call(
        _tc1_body,
        grid=(grid,),
        in_specs=[
            pl.BlockSpec((NW, NP), lambda i: (0, 0)),
            pl.BlockSpec((blk, D_IN), lambda i: (i, 0)),
            pl.BlockSpec((D_IN, H), lambda i: (0, 0)),
        ],
        out_specs=[
            pl.BlockSpec((blk, H), lambda i: (i, 0)),
            pl.BlockSpec((blk, 1), lambda i: (i, 0)),
        ],
        out_shape=[
            jax.ShapeDtypeStruct((NP, H), jnp.float32),
            jax.ShapeDtypeStruct((NP, 1), jnp.float32),
        ],
    )(degp, x, w1)


def _tc2_body(s_ref, ptil_ref, dis_ref, w2_ref, b1_ref, qtil_ref):
    dis = dis_ref[...]
    agg = dis * (s_ref[0] + s_ref[1] + ptil_ref[...]) + b1_ref[...]
    h1 = jnp.maximum(agg, 0.0)
    q = jnp.dot(h1, w2_ref[...], preferred_element_type=jnp.float32)
    qtil_ref[...] = q * dis


def _tc2(s1, ptil, dis, w2, b1):
    blk = 2048
    grid = NP // blk
    return pl.pallas_call(
        _tc2_body,
        grid=(grid,),
        in_specs=[
            pl.BlockSpec((NC, blk, H), lambda i: (0, i, 0)),
            pl.BlockSpec((blk, H), lambda i: (i, 0)),
            pl.BlockSpec((blk, 1), lambda i: (i, 0)),
            pl.BlockSpec((H, H), lambda i: (0, 0)),
            pl.BlockSpec((1, H), lambda i: (0, 0)),
        ],
        out_specs=pl.BlockSpec((blk, H), lambda i: (i, 0)),
        out_shape=jax.ShapeDtypeStruct((NP, H), jnp.float32),
    )(s1, ptil, dis, w2, b1)


def _tc3_body(outf_ref, tsum_ref, qtil_ref, dis_ref, b2_ref, wl_ref, bl_ref,
              out_ref, rows_ref):
    tred = jnp.sum(tsum_ref[...], axis=0)
    rows_ref[...] = jnp.zeros((16, H), jnp.float32)
    for j in range(9):
        idx = outf_ref[j]
        m = jnp.int32(j)
        for k in range(j - 1, -1, -1):
            m = jnp.where(outf_ref[k] == idx, jnp.int32(k), m)
        trow = jnp.zeros((1, H), jnp.float32)
        for k in range(j + 1):
            trow = trow + jnp.where(m == k, 1.0, 0.0) * tred[k:k + 1, :]
        qrow = qtil_ref[pl.ds(idx, 1), :]
        d = dis_ref[pl.ds(idx, 1), :]
        h2 = jnp.maximum(d * (trow + qrow) + b2_ref[...], 0.0)
        rows_ref[pl.ds(j, 1), :] = h2
    z = jnp.dot(rows_ref[...], wl_ref[...], preferred_element_type=jnp.float32)
    out_ref[...] = jax.nn.sigmoid(z + bl_ref[...])


def _tc3(outfall, tsum, qtil, dis, b2, wl, bl):
    return pl.pallas_call(
        _tc3_body,
        in_specs=[
            pl.BlockSpec(memory_space=pltpu.SMEM),
            pl.BlockSpec((NW, 16, H), lambda: (0, 0, 0)),
            pl.BlockSpec((NP, H), lambda: (0, 0)),
            pl.BlockSpec((NP, 1), lambda: (0, 0)),
            pl.BlockSpec((1, H), lambda: (0, 0)),
            pl.BlockSpec((H, 1), lambda: (0, 0)),
            pl.BlockSpec((1, 1), lambda: (0, 0)),
        ],
        out_specs=pl.BlockSpec((16, 1), lambda: (0, 0)),
        out_shape=jax.ShapeDtypeStruct((16, 1), jnp.float32),
        scratch_shapes=[pltpu.VMEM((16, H), jnp.float32)],
    )(outfall, tsum, qtil, dis, b2, wl, bl)


def kernel(x, edge_index, outfall_indices, W1, b1, W2, b2, Wl, bl):
    pad = N + (jnp.arange(EP - E, dtype=jnp.int32) % (NP - N))
    src3 = jnp.concatenate([edge_index[0], pad]).reshape(NW, NITER, CH)
    dst3 = jnp.concatenate([edge_index[1], pad]).reshape(NW, NITER, CH)
    zeros = jnp.zeros((NP, H), jnp.float32)
    x = jnp.concatenate([x, jnp.zeros((NP - N, D_IN), jnp.float32)], axis=0)
    outf16 = jnp.concatenate(
        [outfall_indices, jnp.full((7,), -1, jnp.int32)])

    degp, msrc, mj, cnts = _sc_degscan(dst3, src3, outf16)
    ptil, dis = _tc1(degp, x, W1)
    s1 = _sc_edge(src3, dst3, ptil, zeros)
    qtil = _tc2(s1, ptil, dis, W2, b1.reshape(1, H))
    tsum = _sc_sparse(msrc, mj, cnts, qtil)
    out = _tc3(outfall_indices, tsum, qtil, dis, b2.reshape(1, H),
               Wl, bl.reshape(1, 1))
    return out[:9, 0]


# speedup vs baseline: 1.2477x; 1.0280x over previous
import functools

import jax
import jax.numpy as jnp
from jax import lax
from jax.experimental import pallas as pl
from jax.experimental.pallas import tpu as pltpu
from jax.experimental.pallas import tpu_sc as plsc

N = 10000
E = 320000
D_IN = 128
H = 48

NC = 2
NS = 16
NW = NC * NS
CH = 128
NITER = 80
EPT = NITER * CH
EP = NW * EPT
NP = 10240
RPS = NP // NS
LCAP = EPT + 16

_CPARAMS = dict(needs_layout_passes=False, use_tc_tiling_on_sc=False)


def _vmesh():
    return plsc.VectorSubcoreMesh(
        core_axis_name="c", subcore_axis_name="s",
        num_cores=NC, num_subcores=NS)


@functools.lru_cache(maxsize=None)
def _make_sc_degscan():
  @functools.partial(
      pl.kernel,
      out_type=[
          jax.ShapeDtypeStruct((NW, NP), jnp.float32),
          jax.ShapeDtypeStruct((NW, LCAP), jnp.int32),
          jax.ShapeDtypeStruct((NW, LCAP), jnp.int32),
          jax.ShapeDtypeStruct((NW, 16), jnp.int32),
      ],
      mesh=_vmesh(),
      compiler_params=pltpu.CompilerParams(**_CPARAMS),
      scratch_types=[
          pltpu.VMEM((NITER, CH), jnp.int32),
          pltpu.VMEM((NITER, CH), jnp.int32),
          pltpu.VMEM((NP,), jnp.float32),
          pltpu.VMEM((LCAP,), jnp.int32),
          pltpu.VMEM((LCAP,), jnp.int32),
          pltpu.VMEM((16,), jnp.int32),
          pltpu.VMEM((16,), jnp.int32),
      ],
  )
  def _sc_degscan_k(dst3_hbm, src3_hbm, outf_hbm,
                    degp_hbm, msrc_hbm, mj_hbm, cnt_hbm,
                    dst_v, src_v, deg_v, msrc_v, mj_v, outf_v, cntb_v):
    cid = lax.axis_index("c")
    sid = lax.axis_index("s")
    wid = sid * NC + cid

    def zero_body(i, _):
        deg_v[pl.ds(i * 16, 16)] = jnp.zeros((16,), jnp.float32)
        return 0
    lax.fori_loop(0, NP // 16, zero_body, 0)

    pltpu.sync_copy(dst3_hbm.at[wid], dst_v)
    pltpu.sync_copy(src3_hbm.at[wid], src_v)
    pltpu.sync_copy(outf_hbm, outf_v)

    ones = jnp.ones((16,), jnp.float32)
    lanes = lax.iota(jnp.int32, 16)
    ov = outf_v[...]
    outs = [jnp.sum(jnp.where(lanes == k, ov, 0)) for k in range(9)]

    def body(j, cnt):
        for c in range(CH // 16):
            dstv = dst_v[j, pl.ds(c * 16, 16)]
            plsc.addupdate_scatter(deg_v, [dstv], ones)
            jsel = jnp.full((16,), 15, jnp.int32)
            for k in range(8, -1, -1):
                jsel = jnp.where(dstv == outs[k], jnp.int32(k), jsel)
            mask = jsel < 15
            srcv = src_v[j, pl.ds(c * 16, 16)]
            plsc.store_compressed(msrc_v.at[pl.ds(cnt, 16)], srcv, mask=mask)
            plsc.store_compressed(mj_v.at[pl.ds(cnt, 16)], jsel, mask=mask)
            cnt = cnt + jnp.sum(mask.astype(jnp.int32))
        return cnt
    cnt = lax.fori_loop(0, NITER, body, jnp.int32(0))

    msrc_v[pl.ds(cnt, 16)] = jnp.full((16,), NP - 1, jnp.int32)
    cntb_v[...] = jnp.full((16,), cnt, jnp.int32)

    pltpu.sync_copy(deg_v, degp_hbm.at[wid])
    pltpu.sync_copy(msrc_v, msrc_hbm.at[wid])
    pltpu.sync_copy(mj_v, mj_hbm.at[wid])
    pltpu.sync_copy(cntb_v, cnt_hbm.at[wid])

  return _sc_degscan_k


def _sc_degscan(dst3, src3, outf16):
  return _make_sc_degscan()(dst3, src3, outf16)


@functools.lru_cache(maxsize=None)
def _make_sc_edge():
  @functools.partial(
      pl.kernel,
      out_type=jax.ShapeDtypeStruct((NC, NP, H), jnp.float32),
      mesh=_vmesh(),
      compiler_params=pltpu.CompilerParams(**_CPARAMS),
      scratch_types=[
          pltpu.VMEM((NITER, CH), jnp.int32),
          pltpu.VMEM((NITER, CH), jnp.int32),
          pltpu.VMEM((8, CH, H), jnp.float32),
          pltpu.VMEM_SHARED((NP, H), jnp.float32),
      ] + [pltpu.SemaphoreType.DMA] * 16,
  )
  def _sc_edge_k(src3_hbm, dst3_hbm, table_hbm, zeros_hbm, out_hbm,
                 src_v, dst_v, rowsb, acc_sh, *sems):
    cid = lax.axis_index("c")
    sid = lax.axis_index("s")
    wid = sid * NC + cid

    row0 = pl.multiple_of(sid * RPS, 8)
    pltpu.sync_copy(zeros_hbm.at[pl.ds(row0, RPS)],
                    acc_sh.at[pl.ds(row0, RPS)])
    pltpu.sync_copy(src3_hbm.at[wid], src_v)
    pltpu.sync_copy(dst3_hbm.at[wid], dst_v)
    plsc.subcore_barrier()

    def start_g(j, buf, sem):
        pltpu.async_copy(table_hbm.at[src_v.at[j]], buf, sem)

    def wait_g(j, buf, sem):
        pltpu.make_async_copy(table_hbm.at[src_v.at[j]], buf, sem).wait()

    ND = 8
    bufs = [rowsb.at[b] for b in range(ND)]
    gsems = sems[:ND]
    wsems = sems[ND:]
    for b in range(ND):
        start_g(b, bufs[b], gsems[b])

    def body(jj, _):
        js = [jj * ND + b for b in range(ND)]
        scs = []
        for b in range(ND):
            wait_g(js[b], bufs[b], gsems[b])
            scs.append(pltpu.async_copy(
                bufs[b], acc_sh.at[dst_v.at[js[b]]], wsems[b], add=True))
        for b in range(ND):
            scs[b].wait()

            @pl.when(jj != NITER // ND - 1)
            def _():
                start_g(js[b] + ND, bufs[b], gsems[b])
        return 0
    lax.fori_loop(0, NITER // ND, body, 0)

    plsc.subcore_barrier()
    pltpu.sync_copy(acc_sh.at[pl.ds(row0, RPS)],
                    out_hbm.at[cid, pl.ds(row0, RPS)])

  return _sc_edge_k


def _sc_edge(src3, dst3, table, zeros):
  return _make_sc_edge()(src3, dst3, table, zeros)


@functools.lru_cache(maxsize=None)
def _make_sc_sparse():
  @functools.partial(
      pl.kernel,
      out_type=jax.ShapeDtypeStruct((NW, 16, H), jnp.float32),
      mesh=_vmesh(),
      compiler_params=pltpu.CompilerParams(**_CPARAMS),
      scratch_types=[
          pltpu.VMEM((LCAP,), jnp.int32),
          pltpu.VMEM((LCAP,), jnp.int32),
          pltpu.VMEM((16,), jnp.int32),
          pltpu.VMEM((16, H), jnp.float32),
          pltpu.VMEM((16, H), jnp.float32),
          pltpu.SemaphoreType.DMA,
      ],
  )
  def _sc_sparse_k(msrc_hbm, mj_hbm, cnt_hbm, table_hbm, out_hbm,
                   msrc_v, mj_v, cnt_v, buf_v, t_v, sem):
    cid = lax.axis_index("c")
    sid = lax.axis_index("s")
    wid = sid * NC + cid

    pltpu.sync_copy(msrc_hbm.at[wid], msrc_v)
    pltpu.sync_copy(mj_hbm.at[wid], mj_v)
    pltpu.sync_copy(cnt_hbm.at[wid], cnt_v)

    for r in range(16):
        for m in range(H // 16):
            t_v[r, pl.ds(m * 16, 16)] = jnp.zeros((16,), jnp.float32)

    cnt = jnp.max(cnt_v[...])
    nch = (cnt + 15) // 16
    lanes = lax.iota(jnp.int32, 16)

    def body(c, _):
        base = pl.multiple_of(c * 16, 8)
        pltpu.async_copy(
            table_hbm.at[msrc_v.at[pl.ds(base, 16)]], buf_v, sem).wait()
        jv = mj_v[pl.ds(base, 16)]
        valid = (lanes + base) < cnt
        for m in range(H):
            col = jnp.full((16,), m, jnp.int32)
            vm = plsc.load_gather(buf_v, [lanes, col])
            plsc.addupdate_scatter(t_v, [jv, col], vm, mask=valid)
        return 0
    lax.fori_loop(0, nch, body, 0)

    pltpu.sync_copy(t_v, out_hbm.at[wid])

  return _sc_sparse_k


def _sc_sparse(msrc, mj, cnts, table):
  return _make_sc_sparse()(msrc, mj, cnts, table)


def _tc1_body(degp_ref, x_ref, w1_ref, ptil_ref, dis_ref):
    i = pl.program_id(0)
    blk = x_ref.shape[0]
    deg = jnp.sum(degp_ref[:, pl.ds(i * blk, blk)], axis=0) + 1.0
    dis = lax.rsqrt(deg)
    p = jnp.dot(x_ref[...], w1_ref[...], preferred_element_type=jnp.float32)
    ptil_ref[...] = p * dis[:, None]
    dis_ref[...] = dis[:, None]


def _tc1(degp, x, w1):
    blk = 2048
    grid = NP // blk
    return pl.pallas_call(
        _tc1_body,
        grid=(grid,),
        in_specs=[
            pl.BlockSpec((NW, NP), lambda i: (0, 0)),
            pl.BlockSpec((blk, D_IN), lambda i: (i, 0)),
            pl.BlockSpec((D_IN, H), lambda i: (0, 0)),
        ],
        out_specs=[
            pl.BlockSpec((blk, H), lambda i: (i, 0)),
            pl.BlockSpec((blk, 1), lambda i: (i, 0)),
        ],
        out_shape=[
            jax.ShapeDtypeStruct((NP, H), jnp.float32),
            jax.ShapeDtypeStruct((NP, 1), jnp.float32),
        ],
    )(degp, x, w1)


def _tc2_body(s_ref, ptil_ref, dis_ref, w2_ref, b1_ref, qtil_ref):
    dis = dis_ref[...]
    agg = dis * (s_ref[0] + s_ref[1] + ptil_ref[...]) + b1_ref[...]
    h1 = jnp.maximum(agg, 0.0)
    q = jnp.dot(h1, w2_ref[...], preferred_element_type=jnp.float32)
    qtil_ref[...] = q * dis


def _tc2(s1, ptil, dis, w2, b1):
    blk = 2048
    grid = NP // blk
    return pl.pallas_call(
        _tc2_body,
        grid=(grid,),
        in_specs=[
            pl.BlockSpec((NC, blk, H), lambda i: (0, i, 0)),
            pl.BlockSpec((blk, H), lambda i: (i, 0)),
            pl.BlockSpec((blk, 1), lambda i: (i, 0)),
            pl.BlockSpec((H, H), lambda i: (0, 0)),
            pl.BlockSpec((1, H), lambda i: (0, 0)),
        ],
        out_specs=pl.BlockSpec((blk, H), lambda i: (i, 0)),
        out_shape=jax.ShapeDtypeStruct((NP, H), jnp.float32),
    )(s1, ptil, dis, w2, b1)


def _tc3_body(outf_ref, tsum_ref, qtil_ref, dis_ref, b2_ref, wl_ref, bl_ref,
              out_ref, rows_ref):
    tred = jnp.sum(tsum_ref[...], axis=0)
    rows_ref[...] = jnp.zeros((16, H), jnp.float32)
    for j in range(9):
        idx = outf_ref[j]
        m = jnp.int32(j)
        for k in range(j - 1, -1, -1):
            m = jnp.where(outf_ref[k] == idx, jnp.int32(k), m)
        trow = jnp.zeros((1, H), jnp.float32)
        for k in range(j + 1):
            trow = trow + jnp.where(m == k, 1.0, 0.0) * tred[k:k + 1, :]
        qrow = qtil_ref[pl.ds(idx, 1), :]
        d = dis_ref[pl.ds(idx, 1), :]
        h2 = jnp.maximum(d * (trow + qrow) + b2_ref[...], 0.0)
        rows_ref[pl.ds(j, 1), :] = h2
    z = jnp.dot(rows_ref[...], wl_ref[...], preferred_element_type=jnp.float32)
    out_ref[...] = jax.nn.sigmoid(z + bl_ref[...])


def _tc3(outfall, tsum, qtil, dis, b2, wl, bl):
    return pl.pallas_call(
        _tc3_body,
        in_specs=[
            pl.BlockSpec(memory_space=pltpu.SMEM),
            pl.BlockSpec((NW, 16, H), lambda: (0, 0, 0)),
            pl.BlockSpec((NP, H), lambda: (0, 0)),
            pl.BlockSpec((NP, 1), lambda: (0, 0)),
            pl.BlockSpec((1, H), lambda: (0, 0)),
            pl.BlockSpec((H, 1), lambda: (0, 0)),
            pl.BlockSpec((1, 1), lambda: (0, 0)),
        ],
        out_specs=pl.BlockSpec((16, 1), lambda: (0, 0)),
        out_shape=jax.ShapeDtypeStruct((16, 1), jnp.float32),
        scratch_shapes=[pltpu.VMEM((16, H), jnp.float32)],
    )(outfall, tsum, qtil, dis, b2, wl, bl)


def kernel(x, edge_index, outfall_indices, W1, b1, W2, b2, Wl, bl):
    pad = N + (jnp.arange(EP - E, dtype=jnp.int32) % (NP - N))
    src3 = jnp.concatenate([edge_index[0], pad]).reshape(NW, NITER, CH)
    dst3 = jnp.concatenate([edge_index[1], pad]).reshape(NW, NITER, CH)
    zeros = jnp.zeros((NP, H), jnp.float32)
    x = jnp.concatenate([x, jnp.zeros((NP - N, D_IN), jnp.float32)], axis=0)
    outf16 = jnp.concatenate(
        [outfall_indices, jnp.full((7,), -1, jnp.int32)])

    degp, msrc, mj, cnts = _sc_degscan(dst3, src3, outf16)
    ptil, dis = _tc1(degp, x, W1)
    s1 = _sc_edge(src3, dst3, ptil, zeros)
    qtil = _tc2(s1, ptil, dis, W2, b1.reshape(1, H))
    tsum = _sc_sparse(msrc, mj, cnts, qtil)
    out = _tc3(outfall_indices, tsum, qtil, dis, b2.reshape(1, H),
               Wl, bl.reshape(1, 1))
    return out[:9, 0]
